# speedup vs baseline: 70.1654x; 70.1654x over previous
import jax
import jax.numpy as jnp
from jax.experimental import pallas as pl

_D = 256
_HID = 256
_HEADS = 4
_NL = 96
_EL = 1024
_L = 96
_NH = 64
_EH = 512
_C2 = _HID // 2
_F32 = jnp.float32


def _ln(x, g, b):
    mu = jnp.mean(x, axis=-1, keepdims=True)
    xc = x - mu
    var = jnp.mean(xc * xc, axis=-1, keepdims=True)
    return xc / jnp.sqrt(var + 1e-5) * g + b


def _dot(a, b):
    return jax.lax.dot_general(a, b, (((1,), (0,)), ((), ())),
                               preferred_element_type=_F32)


def _dot_t(a, b):
    return jax.lax.dot_general(a, b, (((1,), (1,)), ((), ())),
                               preferred_element_type=_F32)


def _cnt_from_edges(src_row, dst_row, n):
    e = src_row.shape[1]
    ids = jax.lax.broadcasted_iota(jnp.int32, (n, e), 0)
    s_oh = (src_row == ids).astype(_F32)
    d_oh = (dst_row == ids).astype(_F32)
    cnt = _dot_t(d_oh, s_oh)
    r = jax.lax.broadcasted_iota(jnp.int32, (n, n), 0)
    c = jax.lax.broadcasted_iota(jnp.int32, (n, n), 1)
    return cnt + (r == c).astype(_F32)


def _dense_gat(hw, cnt, a_s, a_d, bias, heads, ch):
    mask = cnt > 0.0
    outs = []
    for k in range(heads):
        hk = hw[:, k * ch:(k + 1) * ch]
        asrc = _dot_t(a_s[k:k + 1, :], hk)
        adst = _dot_t(hk, a_d[k:k + 1, :])
        logits = adst + asrc
        logits = jnp.where(logits >= 0.0, logits, 0.2 * logits)
        lm = jnp.where(mask, logits, -1e30)
        amax = jnp.max(lm, axis=1, keepdims=True)
        ex = cnt * jnp.exp(lm - amax)
        denom = jnp.sum(ex, axis=1, keepdims=True)
        w = ex / (denom + 1e-16)
        outs.append(_dot(w, hk))
    return jnp.concatenate(outs, axis=1) + bias


def _head_kernel(hx_ref, he_ref,
                 few_ref, feb_ref, feg_ref, febt_ref,
                 g1w_ref, g1s_ref, g1d_ref, g1b_ref,
                 g2w_ref, g2s_ref, g2d_ref, g2b_ref,
                 haw_ref, hab_ref, hag_ref, habt_ref,
                 out_ref):
    hx = hx_ref[0]
    src = he_ref[0, 0:1, :]
    dst = he_ref[0, 1:2, :]
    cnt = _cnt_from_edges(src, dst, _NH)
    h = jnp.maximum(_ln(_dot(hx, few_ref[...]) + feb_ref[...],
                        feg_ref[...], febt_ref[...]), 0.0)
    h = jnp.maximum(_dense_gat(_dot(h, g1w_ref[...]), cnt, g1s_ref[...],
                               g1d_ref[...], g1b_ref[...], _HEADS, _C2), 0.0)
    h = jnp.maximum(_dense_gat(_dot(h, g2w_ref[...]), cnt, g2s_ref[...],
                               g2d_ref[...], g2b_ref[...], _HEADS, _C2), 0.0)
    pooled = jnp.mean(h, axis=0, keepdims=True)
    hf = jnp.maximum(_ln(_dot(pooled, haw_ref[...]) + hab_ref[...],
                         hag_ref[...], habt_ref[...]), 0.0)
    out_ref[0] = hf


def _layer_kernel(lx_ref, hf_ref, le_ref,
                  few_ref, feb_ref, feg_ref, febt_ref,
                  lew_ref, leb_ref, leg_ref, lebt_ref,
                  g1w_ref, g1s_ref, g1d_ref, g1b_ref,
                  g2w_ref, g2s_ref, g2d_ref, g2b_ref,
                  g3w_ref, g3s_ref, g3d_ref, g3b_ref,
                  gaw_ref, gab_ref, gag_ref, gabt_ref,
                  o1w_ref, o1b_ref, o1g_ref, o1bt_ref,
                  o2w_ref, o2b_ref,
                  out_ref):
    lx = jnp.maximum(_ln(_dot(lx_ref[...], few_ref[...]) + feb_ref[...],
                         feg_ref[...], febt_ref[...]), 0.0)
    combined = jnp.concatenate([lx, hf_ref[...]], axis=1)
    x = jnp.maximum(_ln(_dot(combined, lew_ref[...]) + leb_ref[...],
                        leg_ref[...], lebt_ref[...]), 0.0)
    src = le_ref[0:1, :]
    dst = le_ref[1:2, :]
    cnt = _cnt_from_edges(src, dst, _NL)
    x = jnp.maximum(_dense_gat(_dot(x, g1w_ref[...]), cnt, g1s_ref[...],
                               g1d_ref[...], g1b_ref[...], _HEADS, _HID), 0.0)
    x = jnp.maximum(_dense_gat(_dot(x, g2w_ref[...]), cnt, g2s_ref[...],
                               g2d_ref[...], g2b_ref[...], _HEADS, _HID), 0.0)
    x = jnp.maximum(_dense_gat(_dot(x, g3w_ref[...]), cnt, g3s_ref[...],
                               g3d_ref[...], g3b_ref[...], _HEADS, _HID), 0.0)
    g = jnp.mean(x, axis=0, keepdims=True)
    g = jnp.maximum(_ln(_dot(g, gaw_ref[...]) + gab_ref[...],
                        gag_ref[...], gabt_ref[...]), 0.0)
    g = jnp.maximum(_ln(_dot(g, o1w_ref[...]) + o1b_ref[...],
                        o1g_ref[...], o1bt_ref[...]), 0.0)
    out_ref[...] = jax.nn.sigmoid(_dot(g, o2w_ref[...]) + o2b_ref[...])


def _row(v):
    return v.reshape(1, -1).astype(_F32)


def kernel(layer_x, layer_edge_index, head_x, head_edge_index, params):
    p = params
    head_edge = head_edge_index.astype(jnp.int32)
    layer_edge = layer_edge_index.astype(jnp.int32)

    head_weights = (
        p['fe_W'], _row(p['fe_b']), _row(p['fe_g']), _row(p['fe_beta']),
        p['hg1_W'], p['hg1_as'], p['hg1_ad'], _row(p['hg1_b']),
        p['hg2_W'], p['hg2_as'], p['hg2_ad'], _row(p['hg2_b']),
        p['ha_W'], _row(p['ha_b']), _row(p['ha_g']), _row(p['ha_beta']),
    )

    def _full(a):
        nd = a.ndim
        return pl.BlockSpec(a.shape, lambda i, _n=nd: (0,) * _n)

    head_feats = pl.pallas_call(
        _head_kernel,
        grid=(_L,),
        in_specs=[
            pl.BlockSpec((1, _NH, _D), lambda i: (i, 0, 0)),
            pl.BlockSpec((1, 2, _EH), lambda i: (i, 0, 0)),
        ] + [_full(w) for w in head_weights],
        out_specs=pl.BlockSpec((1, 1, _HID), lambda i: (i, 0, 0)),
        out_shape=jax.ShapeDtypeStruct((_L, 1, _HID), _F32),
    )(head_x, head_edge, *head_weights)
    head_feats = head_feats.reshape(_L, _HID)

    layer_weights = (
        p['fe_W'], _row(p['fe_b']), _row(p['fe_g']), _row(p['fe_beta']),
        p['le_W'], _row(p['le_b']), _row(p['le_g']), _row(p['le_beta']),
        p['lg1_W'], p['lg1_as'], p['lg1_ad'], _row(p['lg1_b']),
        p['lg2_W'], p['lg2_as'], p['lg2_ad'], _row(p['lg2_b']),
        p['lg3_W'], p['lg3_as'], p['lg3_ad'], _row(p['lg3_b']),
        p['ga_W'], _row(p['ga_b']), _row(p['ga_g']), _row(p['ga_beta']),
        p['o1_W'], _row(p['o1_b']), _row(p['o1_g']), _row(p['o1_beta']),
        p['o2_W'], _row(p['o2_b']),
    )

    out = pl.pallas_call(
        _layer_kernel,
        out_shape=jax.ShapeDtypeStruct((1, 1), _F32),
    )(layer_x, head_feats, layer_edge, *layer_weights)
    return out.reshape((1,))
